# Initial kernel scaffold; baseline (speedup 1.0000x reference)
import jax
import jax.numpy as jnp
from jax.experimental import pallas as pl


def kernel(x, up_W, up_b, down_W, down_b, gate_W, gate_b):
    raise NotImplementedError("write your pallas kernel here")


# speedup vs baseline: 1.2703x; 1.2703x over previous
import functools

import jax
import jax.numpy as jnp
from jax.experimental import pallas as pl
from jax.experimental.pallas import tpu as pltpu


def _ffn_body(x_ref, upW_ref, upb_ref, downW_ref, downb_ref,
              gateW_ref, gateb_ref, y_ref, gate_ref, oh_s, acc_s,
              *, num_f, chunks_per_tile, out_tile, num_tiles):
    f = pl.program_id(1)
    bt = x_ref.shape[0]
    lanes = gate_ref.shape[1]

    @pl.when(f == 0)
    def _router():
        x = x_ref[...]
        logits = jax.lax.dot_general(
            x, gateW_ref[...], (((1,), (1,)), ((), ())),
            preferred_element_type=jnp.float32)
        logits = logits + gateb_ref[...]
        cols = jax.lax.broadcasted_iota(jnp.int32, (bt, lanes), 1)
        neg = jnp.float32(-3e38)
        logits = jnp.where(cols < num_tiles, logits, neg)
        m = jnp.max(logits, axis=1, keepdims=True)
        hit = logits >= m
        first = jnp.min(jnp.where(hit, cols, jnp.int32(lanes)),
                        axis=1, keepdims=True)
        oh = (cols == first).astype(jnp.float32)
        oh_s[...] = oh
        gate_ref[...] = oh

    x = x_ref[...]
    h = jax.lax.dot_general(x, upW_ref[...], (((1,), (1,)), ((), ())),
                            preferred_element_type=jnp.float32)
    h = jnp.maximum(h + upb_ref[0], 0.0)
    oh = oh_s[...]
    cols = jax.lax.broadcasted_iota(jnp.int32, (bt, lanes), 1)
    tile_of_f = f // chunks_per_tile
    g = jnp.sum(oh * (cols == tile_of_f).astype(jnp.float32),
                axis=1, keepdims=True)
    h = h * g

    contrib = jax.lax.dot_general(h, downW_ref[...], (((1,), (1,)), ((), ())),
                                  preferred_element_type=jnp.float32)

    @pl.when(f == 0)
    def _init():
        acc_s[...] = contrib

    @pl.when(f > 0)
    def _acc():
        acc_s[...] += contrib

    @pl.when(f == num_f - 1)
    def _final():
        y = acc_s[...] + downb_ref[...]
        oh2 = oh_s[...]
        colsf = jax.lax.broadcasted_iota(jnp.int32, (bt, lanes), 1)
        idx = jnp.sum(oh2 * colsf.astype(jnp.float32), axis=1, keepdims=True)
        ocols = jax.lax.broadcasted_iota(jnp.int32, y.shape, 1)
        otile = (ocols // out_tile).astype(jnp.float32)
        y_ref[...] = y * (otile == idx).astype(jnp.float32)


def kernel(x, up_W, up_b, down_W, down_b, gate_W, gate_b):
    Bb, Tt, C = x.shape
    N = Bb * Tt
    d_ff = up_W.shape[0]
    num_tiles = gate_W.shape[0]
    out_tile = C // num_tiles

    BT = min(512, N)
    num_f = 4 * num_tiles
    F = d_ff // num_f
    chunks_per_tile = num_f // num_tiles
    LANES = 128

    xf = x.reshape(N, C)
    gW = jnp.zeros((LANES, C), jnp.float32).at[:num_tiles].set(gate_W)
    gb = jnp.zeros((1, LANES), jnp.float32).at[0, :num_tiles].set(gate_b)
    upb2 = up_b.reshape(num_f, 1, F)
    downb2 = down_b.reshape(1, C)

    grid = (N // BT, num_f)
    y, gate = pl.pallas_call(
        functools.partial(_ffn_body, num_f=num_f,
                          chunks_per_tile=chunks_per_tile,
                          out_tile=out_tile, num_tiles=num_tiles),
        grid=grid,
        in_specs=[
            pl.BlockSpec((BT, C), lambda t, f: (t, 0)),
            pl.BlockSpec((F, C), lambda t, f: (f, 0)),
            pl.BlockSpec((1, 1, F), lambda t, f: (f, 0, 0)),
            pl.BlockSpec((C, F), lambda t, f: (0, f)),
            pl.BlockSpec((1, C), lambda t, f: (0, 0)),
            pl.BlockSpec((LANES, C), lambda t, f: (0, 0)),
            pl.BlockSpec((1, LANES), lambda t, f: (0, 0)),
        ],
        out_specs=[
            pl.BlockSpec((BT, C), lambda t, f: (t, 0)),
            pl.BlockSpec((BT, LANES), lambda t, f: (t, 0)),
        ],
        out_shape=[
            jax.ShapeDtypeStruct((N, C), jnp.float32),
            jax.ShapeDtypeStruct((N, LANES), jnp.float32),
        ],
        scratch_shapes=[
            pltpu.VMEM((BT, LANES), jnp.float32),
            pltpu.VMEM((BT, C), jnp.float32),
        ],
    )(xf, up_W, upb2, down_W, downb2, gW, gb)

    out = y.reshape(Bb, Tt, C)
    gate_out = gate[:, :num_tiles].reshape(Bb, Tt, num_tiles)
    return (out, gate_out)


# speedup vs baseline: 3.1877x; 2.5094x over previous
import functools

import jax
import jax.numpy as jnp
from jax import lax
from jax.experimental import pallas as pl
from jax.experimental.pallas import tpu as pltpu
from jax.experimental.pallas import tpu_sc as plsc

LANES = 128


def _router_body(x_ref, gW_ref, gb_ref, oh_ref, pos_ref, cnt_ref,
                 oh_all, run_s, base_s, *, num_tiles, num_t, bt_blk, bmm):
    p = pl.program_id(0)
    t = pl.program_id(1)
    bt = oh_ref.shape[0]
    hp = jax.lax.Precision.HIGHEST

    @pl.when(p == 0)
    def _pass0():
        @pl.when(t == 0)
        def _init():
            run_s[...] = jnp.zeros_like(run_s)

        logits = jax.lax.dot_general(
            x_ref[...], gW_ref[...], (((1,), (1,)), ((), ())),
            preferred_element_type=jnp.float32) + gb_ref[...]
        cols = jax.lax.broadcasted_iota(jnp.int32, (bt, LANES), 1)
        logits = jnp.where(cols < num_tiles, logits, jnp.float32(-3e38))
        m = jnp.max(logits, axis=1, keepdims=True)
        first = jnp.min(jnp.where(logits >= m, cols, jnp.int32(LANES)),
                        axis=1, keepdims=True)
        oh = (cols == first).astype(jnp.float32)
        oh_ref[...] = oh
        pos_ref[...] = jnp.zeros((bt, LANES), jnp.int32)
        oh_all[pl.ds(t * bt_blk, bt_blk), :] = oh
        run_s[...] += jnp.sum(oh, axis=0, keepdims=True)

        @pl.when(t == num_t - 1)
        def _fin():
            cnt = run_s[...].astype(jnp.int32)
            cnt_ref[...] = cnt
            capt = (((cnt + (bmm - 1)) // bmm) * bmm).astype(jnp.float32)
            r = jax.lax.broadcasted_iota(jnp.int32, (LANES, LANES), 0)
            c = jax.lax.broadcasted_iota(jnp.int32, (LANES, LANES), 1)
            triu = (r < c).astype(jnp.float32)
            base_s[...] = jax.lax.dot_general(
                capt, triu, (((1,), (0,)), ((), ())),
                preferred_element_type=jnp.float32, precision=hp)

    @pl.when(p == 1)
    def _pass1():
        @pl.when(t == 0)
        def _init():
            run_s[...] = base_s[...]

        oh = oh_all[pl.ds(t * bt_blk, bt_blk), :]
        oh_ref[...] = oh
        r = jax.lax.broadcasted_iota(jnp.int32, (bt, bt), 0)
        c = jax.lax.broadcasted_iota(jnp.int32, (bt, bt), 1)
        tri = (r > c).astype(jnp.float32)
        ecs = jax.lax.dot_general(tri, oh, (((1,), (0,)), ((), ())),
                                  preferred_element_type=jnp.float32,
                                  precision=hp)
        pos = jnp.sum((ecs + run_s[...]) * oh, axis=1, keepdims=True)
        pos_ref[...] = jnp.broadcast_to(pos.astype(jnp.int32), (bt, LANES))
        run_s[...] += jnp.sum(oh, axis=0, keepdims=True)


def _router(xf, gate_W, gate_b, num_tiles, bmm):
    N, C = xf.shape
    BT = 512
    num_t = N // BT
    gW = jnp.zeros((LANES, C), jnp.float32).at[:num_tiles].set(gate_W)
    gb = jnp.zeros((1, LANES), jnp.float32).at[0, :num_tiles].set(gate_b)
    return pl.pallas_call(
        functools.partial(_router_body, num_tiles=num_tiles, num_t=num_t,
                          bt_blk=BT, bmm=bmm),
        grid=(2, num_t),
        in_specs=[
            pl.BlockSpec((BT, C), lambda p, t: (t * (1 - p), 0)),
            pl.BlockSpec((LANES, C), lambda p, t: (0, 0)),
            pl.BlockSpec((1, LANES), lambda p, t: (0, 0)),
        ],
        out_specs=[
            pl.BlockSpec((BT, LANES), lambda p, t: (t, 0)),
            pl.BlockSpec((BT, LANES), lambda p, t: (t, 0)),
            pl.BlockSpec((1, LANES), lambda p, t: (0, 0)),
        ],
        out_shape=[
            jax.ShapeDtypeStruct((N, LANES), jnp.float32),
            jax.ShapeDtypeStruct((N, LANES), jnp.int32),
            jax.ShapeDtypeStruct((1, LANES), jnp.int32),
        ],
        scratch_shapes=[
            pltpu.VMEM((N, LANES), jnp.float32),
            pltpu.VMEM((1, LANES), jnp.float32),
            pltpu.VMEM((1, LANES), jnp.float32),
        ],
    )(xf, gW, gb)


def _mm_body(bexp_ref, xs_ref, upW_ref, upb_ref, dW_ref, db_ref, y_ref,
             *, out_tile):
    i = pl.program_id(0)
    e = bexp_ref[i]
    h = jax.lax.dot_general(xs_ref[...], upW_ref[0],
                            (((1,), (1,)), ((), ())),
                            preferred_element_type=jnp.float32)
    h = jnp.maximum(h + upb_ref[0], 0.0)
    y = jax.lax.dot_general(h, dW_ref[0],
                            (((1,), (1,)), ((), ())),
                            preferred_element_type=jnp.float32)
    y = y + db_ref[0]
    reps = y_ref.shape[1] // out_tile
    ytile = jnp.concatenate([y] * reps, axis=1)
    ocols = jax.lax.broadcasted_iota(jnp.int32, ytile.shape, 1)
    y_ref[...] = jnp.where((ocols // out_tile) == e, ytile, 0.0)


def _grouped_mm(xs, up_W, up_b, down_W, down_b, bexp, num_tiles, bmm):
    PAD_N, C = xs.shape
    d_ff = up_W.shape[0]
    ftile = d_ff // num_tiles
    out_tile = C // num_tiles
    nblk = PAD_N // bmm
    upW4 = up_W.reshape(num_tiles, ftile, C)
    upb3 = up_b.reshape(num_tiles, 1, ftile)
    dW4 = down_W.reshape(num_tiles, out_tile, num_tiles * ftile)
    db3 = down_b.reshape(num_tiles, 1, out_tile)
    grid_spec = pltpu.PrefetchScalarGridSpec(
        num_scalar_prefetch=1,
        grid=(nblk,),
        in_specs=[
            pl.BlockSpec((bmm, C), lambda i, b: (i, 0)),
            pl.BlockSpec((1, ftile, C), lambda i, b: (b[i], 0, 0)),
            pl.BlockSpec((1, 1, ftile), lambda i, b: (b[i], 0, 0)),
            pl.BlockSpec((1, out_tile, ftile),
                         lambda i, b: (b[i], 0, b[i])),
            pl.BlockSpec((1, 1, out_tile), lambda i, b: (b[i], 0, 0)),
        ],
        out_specs=pl.BlockSpec((bmm, C), lambda i, b: (i, 0)),
    )
    return pl.pallas_call(
        functools.partial(_mm_body, out_tile=out_tile),
        grid_spec=grid_spec,
        out_shape=jax.ShapeDtypeStruct((PAD_N, C), jnp.float32),
    )(bexp, xs, upW4, upb3, dW4, db3)


def _sc_mesh():
    info = plsc.get_sparse_core_info()
    return plsc.VectorSubcoreMesh(core_axis_name="c", subcore_axis_name="s"), \
        info.num_cores, info.num_subcores


def _dispatch(xf, pos, pad_n):
    N, C = xf.shape
    mesh, nc, ns = _sc_mesh()
    per_w = N // (nc * ns)
    nchunk = per_w // 16

    @functools.partial(
        pl.kernel, mesh=mesh,
        out_type=jax.ShapeDtypeStruct((pad_n, C), jnp.float32),
        scratch_types=[
            pltpu.VMEM((16,), jnp.int32),
            pltpu.VMEM((16, C), jnp.float32),
            pltpu.SemaphoreType.DMA,
        ],
    )
    def k(xf_h, pos_h, xs_h, pos_v, xbuf, sem):
        wid = lax.axis_index("s") * nc + lax.axis_index("c")

        def chunk(j, _):
            n0 = wid * per_w + j * 16
            pltpu.sync_copy(pos_h.at[pl.ds(n0, 16)], pos_v)
            pltpu.sync_copy(xf_h.at[pl.ds(n0, 16)], xbuf)
            pltpu.async_copy(xbuf, xs_h.at[pos_v], sem).wait()
            return ()

        lax.fori_loop(0, nchunk, chunk, (), unroll=False)

    return k(xf, pos)


def _combine(y_full, pos, n_out):
    PAD_N, C = y_full.shape
    mesh, nc, ns = _sc_mesh()
    per_w = n_out // (nc * ns)
    nchunk = per_w // 16

    @functools.partial(
        pl.kernel, mesh=mesh,
        out_type=jax.ShapeDtypeStruct((n_out, C), jnp.float32),
        scratch_types=[
            pltpu.VMEM((16,), jnp.int32),
            pltpu.VMEM((16, C), jnp.float32),
            pltpu.SemaphoreType.DMA,
        ],
    )
    def k(y_h, pos_h, out_h, pos_v, ybuf, sem):
        wid = lax.axis_index("s") * nc + lax.axis_index("c")

        def chunk(j, _):
            n0 = wid * per_w + j * 16
            pltpu.sync_copy(pos_h.at[pl.ds(n0, 16)], pos_v)
            pltpu.async_copy(y_h.at[pos_v], ybuf, sem).wait()
            pltpu.sync_copy(ybuf, out_h.at[pl.ds(n0, 16)])
            return ()

        lax.fori_loop(0, nchunk, chunk, (), unroll=False)

    return k(y_full, pos)


def kernel(x, up_W, up_b, down_W, down_b, gate_W, gate_b):
    Bb, Tt, C = x.shape
    N = Bb * Tt
    num_tiles = gate_W.shape[0]
    BMM = 256
    nblk = N // BMM + num_tiles
    PAD_N = nblk * BMM

    xf = x.reshape(N, C)

    oh, pos2, cnt2 = _router(xf, gate_W, gate_b, num_tiles, BMM)
    pos = pos2[:, 0]
    counts = cnt2[0, :num_tiles]

    caps = (counts + BMM - 1) // BMM
    starts = jnp.concatenate([jnp.zeros((1,), jnp.int32),
                              jnp.cumsum(caps)[:-1].astype(jnp.int32)])
    blk_ids = jnp.arange(nblk, dtype=jnp.int32)
    bexp = jnp.sum(blk_ids[None, :] >= starts[1:, None], axis=0,
                   dtype=jnp.int32)

    xs = _dispatch(xf, pos, PAD_N)
    y_full = _grouped_mm(xs, up_W, up_b, down_W, down_b, bexp,
                         num_tiles, BMM)
    out = _combine(y_full, pos, N)

    gate_out = oh[:, :num_tiles].reshape(Bb, Tt, num_tiles)
    return (out.reshape(Bb, Tt, C), gate_out)
